# baseline (speedup 1.0000x reference)
import functools

import jax
import jax.numpy as jnp
from jax import lax
from jax.experimental import pallas as pl
from jax.experimental.pallas import tpu as pltpu
from jax.experimental.pallas import tpu_sc as plsc

N_NODES = 10000
D = 128
DH = D // 2
NP = N_NODES // 2
NC = 2
NS = 16
B = 32
STRIP = 624
TAIL = N_NODES - NS * STRIP
PSTRIP = 312
PTAIL = NP - NS * PSTRIP
PCHUNKS = (32,) * 9 + (24,)


def _matmul_body(x_ref, w_ref, o_ref):
    o_ref[...] = jnp.dot(x_ref[...], w_ref[...],
                         preferred_element_type=jnp.float32)


def _spmm_body(nb, sup_hbm, row2_hbm, par_hbm, col_hbm, w_hbm, bpk_hbm,
               out_hbm, sup_sh, acc, rows0, rows1, col0, col1,
               row0, row1, par0, par1, w0, w1, sem0, sem1):
    c = lax.axis_index("c")
    s = lax.axis_index("s")
    rows = (rows0, rows1)
    cols = (col0, col1)
    rowsi = (row0, row1)
    pars = (par0, par1)
    ws = (w0, w1)
    sems = (sem0, sem1)

    start = s * STRIP
    pltpu.sync_copy(sup_hbm.at[pl.ds(start, STRIP)],
                    sup_sh.at[pl.ds(start, STRIP)])

    @pl.when(s == NS - 1)
    def _stage_tail():
        pltpu.sync_copy(sup_hbm.at[pl.ds(NS * STRIP, TAIL)],
                        sup_sh.at[pl.ds(NS * STRIP, TAIL)])

    pltpu.sync_copy(bpk_hbm.at[c, pl.ds(0, 8)], bias_v)

    def brow(i, carry):
        for sl in range(8):
            sli = pl.ds(sl * 16, 16)
            rows0[i, sli] = bias_v[0, sli]
        return carry
    lax.fori_loop(0, B, brow, 0)
    pstart = s * PSTRIP
    off = 0
    for sz in PCHUNKS:
        pltpu.sync_copy(rows0.at[pl.ds(0, sz)],
                        acc.at[pl.ds(pstart + off, sz)])
        off += sz

    @pl.when(s == NS - 1)
    def _init_tail():
        pltpu.sync_copy(rows0.at[pl.ds(0, PTAIL)],
                        acc.at[pl.ds(NS * PSTRIP, PTAIL)])
    plsc.subcore_barrier()

    def load_meta(g, b):
        base = (s * nb + g) * B
        pltpu.sync_copy(col_hbm.at[pl.ds(base, B)], cols[b])
        pltpu.sync_copy(w_hbm.at[pl.ds(base, B)], ws[b])
        pltpu.sync_copy(row2_hbm.at[pl.ds(base, B)], rowsi[b])
        pltpu.sync_copy(par_hbm.at[pl.ds(base, B)], pars[b])

    half = c * DH

    def scale_grp(b):
        def body(q, carry):
            wvec = ws[b][pl.ds(q * 16, 16)]
            pvec = pars[b][pl.ds(q * 16, 16)]
            base_e = q * 16
            for j in range(16):
                wv = wvec[j]
                dst = pvec[j] * DH
                other = DH - dst
                e = base_e + j
                for sl in range(4):
                    v = rows[b][e, pl.ds(half + sl * 16, 16)] * wv
                    rows[b][e, pl.ds(dst + sl * 16, 16)] = v
                    rows[b][e, pl.ds(other + sl * 16, 16)] = jnp.zeros(
                        (16,), jnp.float32)
            return carry
        return body

    load_meta(0, 0)
    pltpu.async_copy(sup_sh.at[col0], rows0, sem0)

    def pair_body(t, carry):
        for b in (0, 1):
            g = t * 2 + b
            nxt = 1 - b

            @pl.when(g + 1 < nb)
            def _prefetch():
                load_meta(g + 1, nxt)
                pltpu.async_copy(sup_sh.at[cols[nxt]], rows[nxt], sems[nxt])

            pltpu.make_async_copy(sup_sh.at[cols[b]], rows[b],
                                  sems[b]).wait()
            lax.fori_loop(0, B // 16, scale_grp(b), 0)
            pltpu.sync_copy(rows[b], acc.at[rowsi[b]], add=True)
        return carry

    lax.fori_loop(0, nb // 2, pair_body, 0)
    plsc.subcore_barrier()
    pltpu.sync_copy(acc.at[pl.ds(pstart, PSTRIP)],
                    out_hbm.at[c, pl.ds(pstart, PSTRIP)])

    @pl.when(s == NS - 1)
    def _copy_tail():
        pltpu.sync_copy(acc.at[pl.ds(NS * PSTRIP, PTAIL)],
                        out_hbm.at[c, pl.ds(NS * PSTRIP, PTAIL)])


def kernel(input, edge_index, edge_weight, W, b):
    x = input.astype(jnp.float32)
    n, d_in = x.shape
    e = edge_weight.shape[0]

    bm = 1000
    support = pl.pallas_call(
        _matmul_body,
        grid=(n // bm,),
        in_specs=[pl.BlockSpec((bm, d_in), lambda i: (i, 0)),
                  pl.BlockSpec((d_in, D), lambda i: (0, 0))],
        out_specs=pl.BlockSpec((bm, D), lambda i: (i, 0)),
        out_shape=jax.ShapeDtypeStruct((n, D), jnp.float32),
    )(x, W)

    nb = -(-e // (NS * B))
    nb += nb % 2
    epad = NS * nb * B
    pad = epad - e
    rowi = edge_index[0].astype(jnp.int32)
    row2 = jnp.pad(rowi >> 1, (0, pad))
    par = jnp.pad(rowi & 1, (0, pad))
    col = jnp.pad(edge_index[1].astype(jnp.int32), (0, pad))
    w = jnp.pad(edge_weight.astype(jnp.float32), (0, pad))
    bpk = jnp.broadcast_to(jnp.tile(b.reshape(NC, DH), (1, 2))[:, None, :],
                           (NC, 8, D))

    mesh = plsc.VectorSubcoreMesh(core_axis_name="c", subcore_axis_name="s",
                                  num_cores=NC, num_subcores=NS)
    out_pk = pl.kernel(
        functools.partial(_spmm_body, nb),
        out_type=jax.ShapeDtypeStruct((NC, NP, D), jnp.float32),
        mesh=mesh,
        scratch_types=[
            pltpu.VMEM_SHARED((n, D), jnp.float32),
            pltpu.VMEM_SHARED((NP, D), jnp.float32),
            pltpu.VMEM((B, D), jnp.float32),
            pltpu.VMEM((B, D), jnp.float32),
            pltpu.VMEM((B,), jnp.int32),
            pltpu.VMEM((B,), jnp.int32),
            pltpu.VMEM((B,), jnp.int32),
            pltpu.VMEM((B,), jnp.int32),
            pltpu.VMEM((B,), jnp.int32),
            pltpu.VMEM((B,), jnp.int32),
            pltpu.VMEM((B,), jnp.float32),
            pltpu.VMEM((B,), jnp.float32),
            pltpu.SemaphoreType.DMA,
            pltpu.SemaphoreType.DMA,
        ],
    )(support, row2, par, col, w, bpk)

    halves = out_pk.reshape(NC, n, DH)
    return jnp.concatenate([halves[0], halves[1]], axis=1)


# speedup vs baseline: 1.9491x; 1.9491x over previous
import functools

import jax
import jax.numpy as jnp
from jax import lax
from jax.experimental import pallas as pl
from jax.experimental.pallas import tpu as pltpu
from jax.experimental.pallas import tpu_sc as plsc

N_NODES = 10000
D = 128
DH = D // 2
NP = N_NODES // 2
NC = 2
NS = 16
B = 32
MCH = 4
MWI = MCH * 3 * B
MWW = MCH * B
STRIP = 624
TAIL = N_NODES - NS * STRIP
PSTRIP = 312
PTAIL = NP - NS * PSTRIP
PCHUNKS = (32,) * 9 + (24,)


def _matmul_body(x_ref, w_ref, o_ref):
    o_ref[...] = jnp.dot(x_ref[...], w_ref[...],
                         preferred_element_type=jnp.float32)


def _spmm_body(nb, sup_hbm, meta_hbm, metaw_hbm, bpk_hbm, out_hbm,
               sup_sh, acc, meta0, meta1, mw0, mw1, r2a, r2b, rows0, rows1,
               gsem0, gsem1, msem0, msem1):
    c = lax.axis_index("c")
    s = lax.axis_index("s")
    metas = (meta0, meta1)
    mws = (mw0, mw1)
    r2s = (r2a, r2b)
    rows = (rows0, rows1)
    gsems = (gsem0, gsem1)
    msems = (msem0, msem1)
    zero16 = jnp.zeros((16,), jnp.float32)

    start = s * STRIP
    pltpu.sync_copy(sup_hbm.at[pl.ds(start, STRIP)],
                    sup_sh.at[pl.ds(start, STRIP)])

    @pl.when(s == NS - 1)
    def _stage_tail():
        pltpu.sync_copy(sup_hbm.at[pl.ds(NS * STRIP, TAIL)],
                        sup_sh.at[pl.ds(NS * STRIP, TAIL)])

    pltpu.sync_copy(bpk_hbm.at[c, pl.ds(0, 8)], rows0.at[pl.ds(0, 8)])

    def brow(i, carry):
        for sl in range(8):
            sli = pl.ds(sl * 16, 16)
            rows0[i, sli] = rows0[0, sli]
        return carry
    lax.fori_loop(1, B, brow, 0)
    pstart = s * PSTRIP
    off = 0
    for sz in PCHUNKS:
        pltpu.sync_copy(rows0.at[pl.ds(0, sz)],
                        acc.at[pl.ds(pstart + off, sz)])
        off += sz

    @pl.when(s == NS - 1)
    def _init_tail():
        pltpu.sync_copy(rows0.at[pl.ds(0, PTAIL)],
                        acc.at[pl.ds(NS * PSTRIP, PTAIL)])
    plsc.subcore_barrier()

    nch = nb // MCH
    tile_base = s * nb * 3 * B
    tile_wbase = s * nb * B
    half = c * DH

    def scale_q(mv, mwv, k, b2):
        def body(q, carry):
            koff = k * (3 * B)
            wvec = mwv[pl.ds(k * B + q * 16, 16)]
            pvec = mv[pl.ds(koff + 2 * B + q * 16, 16)]
            for j in range(16):
                wv = wvec[j]
                dst = pvec[j] * DH
                other = DH - dst
                e = q * 16 + j
                for sl in range(4):
                    v = rows[b2][e, pl.ds(half + sl * 16, 16)] * wv
                    rows[b2][e, pl.ds(dst + sl * 16, 16)] = v
                    rows[b2][e, pl.ds(other + sl * 16, 16)] = zero16
            return carry
        return body

    def meta_load(moff, mb):
        pltpu.async_copy(meta_hbm.at[pl.ds(tile_base + moff * MWI, MWI)],
                         metas[mb], msems[mb])
        pltpu.async_copy(metaw_hbm.at[pl.ds(tile_wbase + moff * MWW, MWW)],
                         mws[mb], msems[mb])

    def chunk_run(m, mb):
        mv = metas[mb]
        mwv = mws[mb]
        r2 = r2s[mb]
        pltpu.make_async_copy(meta_hbm.at[pl.ds(tile_base, MWI)],
                              mv, msems[mb]).wait()
        pltpu.make_async_copy(metaw_hbm.at[pl.ds(tile_wbase, MWW)],
                              mwv, msems[mb]).wait()

        @pl.when(m + 1 < nch)
        def _pref_meta():
            meta_load(m + 1, 1 - mb)

        def unp(k, carry):
            for h in range(2):
                r2[k, pl.ds(16 * h, 16)] = mv[pl.ds(k * (3 * B) + B
                                                    + 16 * h, 16)]
            return carry
        lax.fori_loop(0, MCH, unp, 0)

        pltpu.async_copy(sup_sh.at[mv.at[pl.ds(0, B)]], rows[0], gsems[0])

        def pairb(t, carry):
            for b2 in (0, 1):
                k = t * 2 + b2
                nxt = 1 - b2

                @pl.when(k + 1 < MCH)
                def _pref():
                    pltpu.async_copy(
                        sup_sh.at[mv.at[pl.ds((k + 1) * (3 * B), B)]],
                        rows[nxt], gsems[nxt])

                pltpu.make_async_copy(sup_sh.at[mv.at[pl.ds(0, B)]],
                                      rows[b2], gsems[b2]).wait()
                lax.fori_loop(0, B // 16, scale_q(mv, mwv, k, b2), 0)
                pltpu.sync_copy(rows[b2], acc.at[r2.at[k]], add=True)
            return carry

        lax.fori_loop(0, MCH // 2, pairb, 0)

    meta_load(0, 0)

    def pair_chunks(t2, carry):
        chunk_run(t2 * 2, 0)
        chunk_run(t2 * 2 + 1, 1)
        return carry

    lax.fori_loop(0, nch // 2, pair_chunks, 0)
    plsc.subcore_barrier()
    pltpu.sync_copy(acc.at[pl.ds(pstart, PSTRIP)],
                    out_hbm.at[c, pl.ds(pstart, PSTRIP)])

    @pl.when(s == NS - 1)
    def _copy_tail():
        pltpu.sync_copy(acc.at[pl.ds(NS * PSTRIP, PTAIL)],
                        out_hbm.at[c, pl.ds(NS * PSTRIP, PTAIL)])


def kernel(input, edge_index, edge_weight, W, b):
    x = input.astype(jnp.float32)
    n, d_in = x.shape
    e = edge_weight.shape[0]

    bm = 1000
    support = pl.pallas_call(
        _matmul_body,
        grid=(n // bm,),
        in_specs=[pl.BlockSpec((bm, d_in), lambda i: (i, 0)),
                  pl.BlockSpec((d_in, D), lambda i: (0, 0))],
        out_specs=pl.BlockSpec((bm, D), lambda i: (i, 0)),
        out_shape=jax.ShapeDtypeStruct((n, D), jnp.float32),
    )(x, W)

    nb = -(-e // (NS * B))
    nb = -(-nb // (2 * MCH)) * (2 * MCH)
    epad = NS * nb * B
    pad = epad - e
    rowi = edge_index[0].astype(jnp.int32)
    row2 = jnp.pad(rowi >> 1, (0, pad))
    par = jnp.pad(rowi & 1, (0, pad))
    col = jnp.pad(edge_index[1].astype(jnp.int32), (0, pad))
    metaw = jnp.pad(edge_weight.astype(jnp.float32), (0, pad))
    g = epad // B
    meta = jnp.stack([col.reshape(g, B), row2.reshape(g, B),
                      par.reshape(g, B)], axis=1).reshape(-1)
    bpk = jnp.broadcast_to(jnp.tile(b.reshape(NC, DH), (1, 2))[:, None, :],
                           (NC, 8, D))

    mesh = plsc.VectorSubcoreMesh(core_axis_name="c", subcore_axis_name="s",
                                  num_cores=NC, num_subcores=NS)
    out_pk = pl.kernel(
        functools.partial(_spmm_body, nb),
        out_type=jax.ShapeDtypeStruct((NC, NP, D), jnp.float32),
        mesh=mesh,
        scratch_types=[
            pltpu.VMEM_SHARED((n, D), jnp.float32),
            pltpu.VMEM_SHARED((NP, D), jnp.float32),
            pltpu.VMEM((MWI,), jnp.int32),
            pltpu.VMEM((MWI,), jnp.int32),
            pltpu.VMEM((MWW,), jnp.float32),
            pltpu.VMEM((MWW,), jnp.float32),
            pltpu.VMEM((MCH, B), jnp.int32),
            pltpu.VMEM((MCH, B), jnp.int32),
            pltpu.VMEM((B, D), jnp.float32),
            pltpu.VMEM((B, D), jnp.float32),
            pltpu.SemaphoreType.DMA,
            pltpu.SemaphoreType.DMA,
            pltpu.SemaphoreType.DMA,
            pltpu.SemaphoreType.DMA,
        ],
    )(support, meta, metaw, bpk)

    halves = out_pk.reshape(NC, n, DH)
    return jnp.concatenate([halves[0], halves[1]], axis=1)


# speedup vs baseline: 3.1304x; 1.6061x over previous
import functools

import jax
import jax.numpy as jnp
from jax import lax
from jax.experimental import pallas as pl
from jax.experimental.pallas import tpu as pltpu
from jax.experimental.pallas import tpu_sc as plsc

N_NODES = 10000
D = 128
NC = 2
NS = 16
NW = NC * NS
B = 128
NSPLIT = 4
BQ = B // NSPLIT
LANES = 8
STRIP = 624
TAIL = N_NODES - NS * STRIP
ZCHUNKS = (128, 128, 128, 128, 112)


def _matmul_body(x_ref, w_ref, o_ref):
    o_ref[...] = jnp.dot(x_ref[...], w_ref[...],
                         preferred_element_type=jnp.float32)


def _combine_body(p_ref, b_ref, o_ref):
    o_ref[...] = p_ref[0] + p_ref[1] + b_ref[...]


def _spmm_body(nb, sup_hbm, row_hbm, col_hbm, w_hbm, out_hbm,
               acc, rows0, rows1, col0, col1, row0, row1, w0, w1,
               sem0, sem1):
    c = lax.axis_index("c")
    s = lax.axis_index("s")
    wid = s * NC + c
    zero16 = jnp.zeros((16,), jnp.float32)
    rows = (rows0, rows1)
    cols = (col0, col1)
    rowsi = (row0, row1)
    ws = (w0, w1)
    sems = (sem0, sem1)

    def zrow(i, carry):
        for sl in range(LANES):
            rows0[i, pl.ds(sl * 16, 16)] = zero16
        return carry
    lax.fori_loop(0, B, zrow, 0)
    start = s * STRIP
    off = 0
    for sz in ZCHUNKS:
        pltpu.sync_copy(rows0.at[pl.ds(0, sz)],
                        acc.at[pl.ds(start + off, sz)])
        off += sz

    @pl.when(s == NS - 1)
    def _zero_tail():
        pltpu.sync_copy(rows0.at[pl.ds(0, TAIL)],
                        acc.at[pl.ds(NS * STRIP, TAIL)])
    plsc.subcore_barrier()

    def load_meta(g, b):
        base = (wid * nb + g) * B
        pltpu.sync_copy(col_hbm.at[pl.ds(base, B)], cols[b])
        pltpu.sync_copy(w_hbm.at[pl.ds(base, B)], ws[b])
        pltpu.sync_copy(row_hbm.at[pl.ds(base, B)], rowsi[b])

    def gather(b):
        for h in range(NSPLIT):
            pltpu.async_copy(sup_hbm.at[cols[b].at[pl.ds(h * BQ, BQ)]],
                             rows[b].at[pl.ds(h * BQ, BQ)], sems[b])

    def gather_wait(b):
        for h in range(NSPLIT):
            pltpu.make_async_copy(
                sup_hbm.at[cols[b].at[pl.ds(h * BQ, BQ)]],
                rows[b].at[pl.ds(h * BQ, BQ)], sems[b]).wait()

    def scale_grp(b):
        def body(q, carry):
            wvec = ws[b][pl.ds(q * 16, 16)]
            base_e = q * 16
            for j in range(16):
                wv = wvec[j]
                for sl in range(LANES):
                    sli = pl.ds(sl * 16, 16)
                    rows[b][base_e + j, sli] = rows[b][base_e + j, sli] * wv
            return carry
        return body

    load_meta(0, 0)
    gather(0)

    def pair_body(t, carry):
        for b in (0, 1):
            g = t * 2 + b
            nxt = 1 - b

            @pl.when(g + 1 < nb)
            def _prefetch():
                load_meta(g + 1, nxt)
                gather(nxt)

            gather_wait(b)
            lax.fori_loop(0, B // 16, scale_grp(b), 0)
            pltpu.sync_copy(rows[b], acc.at[rowsi[b]], add=True)
        return carry

    lax.fori_loop(0, nb // 2, pair_body, 0)
    plsc.subcore_barrier()
    pltpu.sync_copy(acc.at[pl.ds(start, STRIP)],
                    out_hbm.at[c, pl.ds(start, STRIP)])

    @pl.when(s == NS - 1)
    def _copy_tail():
        pltpu.sync_copy(acc.at[pl.ds(NS * STRIP, TAIL)],
                        out_hbm.at[c, pl.ds(NS * STRIP, TAIL)])


def kernel(input, edge_index, edge_weight, W, b):
    x = input.astype(jnp.float32)
    n, d_in = x.shape
    e = edge_weight.shape[0]

    bm = 1000
    support = pl.pallas_call(
        _matmul_body,
        grid=(n // bm,),
        in_specs=[pl.BlockSpec((bm, d_in), lambda i: (i, 0)),
                  pl.BlockSpec((d_in, D), lambda i: (0, 0))],
        out_specs=pl.BlockSpec((bm, D), lambda i: (i, 0)),
        out_shape=jax.ShapeDtypeStruct((n, D), jnp.float32),
    )(x, W)

    nb = -(-e // (NW * B))
    nb += nb % 2
    epad = NW * nb * B
    pad = epad - e
    row = jnp.pad(edge_index[0].astype(jnp.int32), (0, pad))
    col = jnp.pad(edge_index[1].astype(jnp.int32), (0, pad))
    w = jnp.pad(edge_weight.astype(jnp.float32), (0, pad))

    mesh = plsc.VectorSubcoreMesh(core_axis_name="c", subcore_axis_name="s",
                                  num_cores=NC, num_subcores=NS)
    partials = pl.kernel(
        functools.partial(_spmm_body, nb),
        out_type=jax.ShapeDtypeStruct((NC, n, D), jnp.float32),
        mesh=mesh,
        scratch_types=[
            pltpu.VMEM_SHARED((n, D), jnp.float32),
            pltpu.VMEM((B, D), jnp.float32),
            pltpu.VMEM((B, D), jnp.float32),
            pltpu.VMEM((B,), jnp.int32),
            pltpu.VMEM((B,), jnp.int32),
            pltpu.VMEM((B,), jnp.int32),
            pltpu.VMEM((B,), jnp.int32),
            pltpu.VMEM((B,), jnp.float32),
            pltpu.VMEM((B,), jnp.float32),
            pltpu.SemaphoreType.DMA,
            pltpu.SemaphoreType.DMA,
        ],
    )(support, row, col, w)

    out = pl.pallas_call(
        _combine_body,
        grid=(n // bm,),
        in_specs=[pl.BlockSpec((NC, bm, D), lambda i: (0, i, 0)),
                  pl.BlockSpec((1, D), lambda i: (0, 0))],
        out_specs=pl.BlockSpec((bm, D), lambda i: (i, 0)),
        out_shape=jax.ShapeDtypeStruct((n, D), jnp.float32),
    )(partials, b.reshape(1, D))
    return out
